# Initial kernel scaffold; baseline (speedup 1.0000x reference)
import jax
import jax.numpy as jnp
from jax.experimental import pallas as pl


def kernel(prev_pos, pred_pos, cloth_faces, vertex_type, nodes_from, faces_to, iter_num):
    raise NotImplementedError("write your pallas kernel here")


# speedup vs baseline: 108.0079x; 108.0079x over previous
import functools

import jax
import jax.numpy as jnp
from jax import lax
from jax.experimental import pallas as pl
from jax.experimental.pallas import tpu as pltpu
from jax.experimental.pallas import tpu_sc as plsc

_NC = 2
_NS = 16
_NW = _NC * _NS
_L = 16

_N = 100000
_F = 200000
_C = 1600000

_FA_CHUNK = 640
_FA_ITERS = 10
_F_PAD = _NW * _FA_CHUNK * _FA_ITERS

_CB_CHUNK = 640
_CB_ITERS = 79
_C_PAD = _NW * _CB_CHUNK * _CB_ITERS
_C_PER_TILE = _CB_CHUNK * _CB_ITERS

_HANDLE = 3.0
_PINNED_W = 10.0
_CORR_EPS = 1.0
_PEN_EPS = 0.1


def _rsqrt_nr(x):
    i = lax.bitcast_convert_type(x, jnp.int32)
    i = 0x5F3759DF - (i >> 1)
    y = lax.bitcast_convert_type(i, jnp.float32)
    for _ in range(3):
        y = y * (1.5 - 0.5 * x * y * y)
    return y


def _wid():
    return lax.axis_index("s") * _NC + lax.axis_index("c")


def _face_stage_body(ntab_hbm, f0_hbm, f1_hbm, f2_hbm, ftab_hbm,
                     i0_v, i1_v, i2_v, b0, b1, b2, obuf, sem):
    base0 = _wid() * (_FA_CHUNK * _FA_ITERS)
    lanes = lax.iota(jnp.int32, _L)

    def chunk(j, carry):
        cbase = base0 + j * _FA_CHUNK
        pltpu.sync_copy(f0_hbm.at[pl.ds(cbase, _FA_CHUNK)], i0_v)
        pltpu.sync_copy(f1_hbm.at[pl.ds(cbase, _FA_CHUNK)], i1_v)
        pltpu.sync_copy(f2_hbm.at[pl.ds(cbase, _FA_CHUNK)], i2_v)
        copies = []
        for k in range(_FA_CHUNK // 128):
            sl = pl.ds(k * 128, 128)
            copies.append(pltpu.async_copy(ntab_hbm.at[i0_v.at[sl]], b0.at[sl], sem))
            copies.append(pltpu.async_copy(ntab_hbm.at[i1_v.at[sl]], b1.at[sl], sem))
            copies.append(pltpu.async_copy(ntab_hbm.at[i2_v.at[sl]], b2.at[sl], sem))
        for c in copies:
            c.wait()

        def group(m, carry2):
            rows = m * _L + lanes

            def ld(buf, field):
                return plsc.load_gather(buf, [rows, jnp.full((_L,), field, jnp.int32)])

            x0, y0, z0 = ld(b0, 0), ld(b0, 1), ld(b0, 2)
            t0 = ld(b0, 3)
            u0, v0, w0 = ld(b0, 4), ld(b0, 5), ld(b0, 6)
            x1, y1, z1 = ld(b1, 0), ld(b1, 1), ld(b1, 2)
            t1 = ld(b1, 3)
            u1, v1, w1 = ld(b1, 4), ld(b1, 5), ld(b1, 6)
            x2, y2, z2 = ld(b2, 0), ld(b2, 1), ld(b2, 2)
            t2 = ld(b2, 3)
            u2, v2, w2 = ld(b2, 4), ld(b2, 5), ld(b2, 6)

            def plane(ax, ay, az, bx, by, bz, cx, cy, cz):
                e1x, e1y, e1z = bx - ax, by - ay, bz - az
                e2x, e2y, e2z = cx - ax, cy - ay, cz - az
                nx = e1y * e2z - e1z * e2y
                ny = e1z * e2x - e1x * e2z
                nz = e1x * e2y - e1y * e2x
                nn = nx * nx + ny * ny + nz * nz
                r = _rsqrt_nr(jnp.maximum(nn, 1e-30))
                inv = 1.0 / (nn * r + 1e-12)
                nx, ny, nz = nx * inv, ny * inv, nz * inv
                ch = ax * nx + ay * ny + az * nz
                return nx, ny, nz, ch

            pnx, pny, pnz, pch = plane(x0, y0, z0, x1, y1, z1, x2, y2, z2)
            cnx, cny, cnz, cch = plane(u0, v0, w0, u1, v1, w1, u2, v2, w2)
            pinned = (t0 == _HANDLE) | (t1 == _HANDLE) | (t2 == _HANDLE)
            wf = jnp.where(pinned, _PINNED_W, 1.0)

            for field, val in enumerate((pnx, pny, pnz, pch, cnx, cny, cnz, cch, wf)):
                plsc.store_scatter(
                    obuf, [rows, jnp.full((_L,), field, jnp.int32)], val)
            return carry2

        lax.fori_loop(0, _FA_CHUNK // _L, group, 0)
        pltpu.sync_copy(obuf, ftab_hbm.at[pl.ds(cbase, _FA_CHUNK)])
        return carry

    lax.fori_loop(0, _FA_ITERS, chunk, 0)


def _corr_stage_body(nf_hbm, ft_hbm, ftab_hbm, ntab_hbm, out_hbm,
                     nf_v, ft_v, fbuf, nbuf, acc_v, sem):
    w = _wid()
    base0 = w * _C_PER_TILE
    lanes = lax.iota(jnp.int32, _L)
    acc_v[...] = jnp.zeros((_L,), jnp.float32)

    def chunk(j, carry):
        cbase = base0 + j * _CB_CHUNK
        pltpu.sync_copy(nf_hbm.at[pl.ds(cbase, _CB_CHUNK)], nf_v)
        pltpu.sync_copy(ft_hbm.at[pl.ds(cbase, _CB_CHUNK)], ft_v)
        copies = []
        for k in range(_CB_CHUNK // 128):
            sl = pl.ds(k * 128, 128)
            copies.append(pltpu.async_copy(ftab_hbm.at[ft_v.at[sl]], fbuf.at[sl], sem))
            copies.append(pltpu.async_copy(ntab_hbm.at[nf_v.at[sl]], nbuf.at[sl], sem))
        for c in copies:
            c.wait()

        def group(m, carry2):
            rows = m * _L + lanes

            def ldf(field):
                return plsc.load_gather(
                    fbuf, [rows, jnp.full((_L,), field, jnp.int32)])

            def ldn(field):
                return plsc.load_gather(
                    nbuf, [rows, jnp.full((_L,), field, jnp.int32)])

            pnx, pny, pnz, pch = ldf(0), ldf(1), ldf(2), ldf(3)
            cnx, cny, cnz, cch = ldf(4), ldf(5), ldf(6), ldf(7)
            wf = ldf(8)
            px, py, pz = ldn(0), ldn(1), ldn(2)
            vt = ldn(3)
            cx, cy, cz = ldn(4), ldn(5), ldn(6)

            d_prev = px * pnx + py * pny + pz * pnz - pch
            d_curr = cx * cnx + cy * cny + cz * cnz - cch
            stashed = jnp.abs(d_prev) < _CORR_EPS
            dc = d_curr * jnp.sign(d_prev)
            interp = jnp.maximum(_PEN_EPS - dc, 0.0)
            wn = jnp.where(vt == _HANDLE, _PINNED_W, 1.0)
            t = interp * jnp.maximum(wf, wn)
            valid = (cbase + rows) < _C
            acc_v[...] = acc_v[...] + jnp.where(stashed & valid, t * t * t, 0.0)
            return carry2

        lax.fori_loop(0, _CB_CHUNK // _L, group, 0)
        return carry

    lax.fori_loop(0, _CB_ITERS, chunk, 0)
    pltpu.sync_copy(acc_v, out_hbm.at[w])


def _mesh():
    return plsc.VectorSubcoreMesh(core_axis_name="c", subcore_axis_name="s",
                                  num_cores=_NC, num_subcores=_NS)


_face_stage = functools.partial(
    pl.kernel,
    out_type=jax.ShapeDtypeStruct((_F_PAD, 16), jnp.float32),
    mesh=_mesh(),
    compiler_params=pltpu.CompilerParams(
        needs_layout_passes=False, use_tc_tiling_on_sc=False),
    scratch_types=[
        pltpu.VMEM((_FA_CHUNK,), jnp.int32),
        pltpu.VMEM((_FA_CHUNK,), jnp.int32),
        pltpu.VMEM((_FA_CHUNK,), jnp.int32),
        pltpu.VMEM((_FA_CHUNK, 8), jnp.float32),
        pltpu.VMEM((_FA_CHUNK, 8), jnp.float32),
        pltpu.VMEM((_FA_CHUNK, 8), jnp.float32),
        pltpu.VMEM((_FA_CHUNK, 16), jnp.float32),
        pltpu.SemaphoreType.DMA,
    ],
)(_face_stage_body)

_corr_stage = functools.partial(
    pl.kernel,
    out_type=jax.ShapeDtypeStruct((_NW, _L), jnp.float32),
    mesh=_mesh(),
    compiler_params=pltpu.CompilerParams(
        needs_layout_passes=False, use_tc_tiling_on_sc=False),
    scratch_types=[
        pltpu.VMEM((_CB_CHUNK,), jnp.int32),
        pltpu.VMEM((_CB_CHUNK,), jnp.int32),
        pltpu.VMEM((_CB_CHUNK, 16), jnp.float32),
        pltpu.VMEM((_CB_CHUNK, 8), jnp.float32),
        pltpu.VMEM((_L,), jnp.float32),
        pltpu.SemaphoreType.DMA,
    ],
)(_corr_stage_body)


def kernel(prev_pos, pred_pos, cloth_faces, vertex_type, nodes_from, faces_to, iter_num):
    it = jnp.maximum(iter_num - 0, 0)
    progress = jnp.minimum(it / 1000, 1.0)
    weight = 0.0 + (1.0 - 0.0) * progress

    vt_f = vertex_type.astype(jnp.float32)
    ntab = jnp.concatenate(
        [prev_pos, vt_f, pred_pos, jnp.zeros((_N, 1), jnp.float32)], axis=1)

    f0 = jnp.pad(cloth_faces[:, 0], (0, _F_PAD - _F))
    f1 = jnp.pad(cloth_faces[:, 1], (0, _F_PAD - _F))
    f2 = jnp.pad(cloth_faces[:, 2], (0, _F_PAD - _F))
    nf = jnp.pad(nodes_from[:, 0], (0, _C_PAD - _C))
    ft = jnp.pad(faces_to[:, 0], (0, _C_PAD - _C))

    ftab = _face_stage(ntab, f0, f1, f2)
    partials = _corr_stage(nf, ft, ftab, ntab)

    loss = jnp.sum(partials) * weight
    return (loss, jnp.float32(weight))


# speedup vs baseline: 129.6964x; 1.2008x over previous
import functools

import jax
import jax.numpy as jnp
from jax import lax
from jax.experimental import pallas as pl
from jax.experimental.pallas import tpu as pltpu
from jax.experimental.pallas import tpu_sc as plsc

_NC = 2
_NS = 16
_NW = _NC * _NS
_L = 16

_N = 100000
_F = 200000
_C = 1600000

_FA_CHUNK = 640
_FA_ITERS = 10
_F_PAD = _NW * _FA_CHUNK * _FA_ITERS

_CB_CHUNK = 1280
_CB_ITERS = 40
_C_PAD = _NW * _CB_CHUNK * _CB_ITERS
_C_PER_TILE = _CB_CHUNK * _CB_ITERS

_HANDLE = 3.0
_PINNED_W = 10.0
_CORR_EPS = 1.0
_PEN_EPS = 0.1


def _rsqrt_nr(x):
    i = lax.bitcast_convert_type(x, jnp.int32)
    i = 0x5F3759DF - (i >> 1)
    y = lax.bitcast_convert_type(i, jnp.float32)
    for _ in range(3):
        y = y * (1.5 - 0.5 * x * y * y)
    return y


def _wid():
    return lax.axis_index("s") * _NC + lax.axis_index("c")


def _face_stage_body(ntab_hbm, f0_hbm, f1_hbm, f2_hbm, ftab_hbm,
                     i0a, i1a, i2a, i0b, i1b, i2b,
                     b0a, b1a, b2a, b0b, b1b, b2b, obuf,
                     semIa, semIb, semGa, semGb):
    base0 = _wid() * (_FA_CHUNK * _FA_ITERS)
    lanes = lax.iota(jnp.int32, _L)
    idx = ((i0a, i1a, i2a), (i0b, i1b, i2b))
    gb = ((b0a, b1a, b2a), (b0b, b1b, b2b))
    semI = (semIa, semIb)
    semG = (semGa, semGb)

    def idx_copies(c, b):
        cbase = base0 + c * _FA_CHUNK
        sl = pl.ds(cbase, _FA_CHUNK)
        return [pltpu.make_async_copy(f0_hbm.at[sl], idx[b][0], semI[b]),
                pltpu.make_async_copy(f1_hbm.at[sl], idx[b][1], semI[b]),
                pltpu.make_async_copy(f2_hbm.at[sl], idx[b][2], semI[b])]

    def gath_copies(b):
        cs = []
        for k in range(_FA_CHUNK // 128):
            sl = pl.ds(k * 128, 128)
            for v in range(3):
                cs.append(pltpu.make_async_copy(
                    ntab_hbm.at[idx[b][v].at[sl]], gb[b][v].at[sl], semG[b]))
        return cs

    def clamp(c):
        return jnp.minimum(c, _FA_ITERS - 1)

    def compute(b, cbase):
        b0, b1, b2 = gb[b]

        def group(m, carry2):
            rows = m * _L + lanes

            def ld(buf, field):
                return plsc.load_gather(buf, [rows, jnp.full((_L,), field, jnp.int32)])

            x0, y0, z0 = ld(b0, 0), ld(b0, 1), ld(b0, 2)
            t0 = ld(b0, 3)
            u0, v0, w0 = ld(b0, 4), ld(b0, 5), ld(b0, 6)
            x1, y1, z1 = ld(b1, 0), ld(b1, 1), ld(b1, 2)
            t1 = ld(b1, 3)
            u1, v1, w1 = ld(b1, 4), ld(b1, 5), ld(b1, 6)
            x2, y2, z2 = ld(b2, 0), ld(b2, 1), ld(b2, 2)
            t2 = ld(b2, 3)
            u2, v2, w2 = ld(b2, 4), ld(b2, 5), ld(b2, 6)

            def plane(ax, ay, az, bx, by, bz, cx, cy, cz):
                e1x, e1y, e1z = bx - ax, by - ay, bz - az
                e2x, e2y, e2z = cx - ax, cy - ay, cz - az
                nx = e1y * e2z - e1z * e2y
                ny = e1z * e2x - e1x * e2z
                nz = e1x * e2y - e1y * e2x
                nn = nx * nx + ny * ny + nz * nz
                r = _rsqrt_nr(jnp.maximum(nn, 1e-30))
                inv = 1.0 / (nn * r + 1e-12)
                nx, ny, nz = nx * inv, ny * inv, nz * inv
                ch = ax * nx + ay * ny + az * nz
                return nx, ny, nz, ch

            pnx, pny, pnz, pch = plane(x0, y0, z0, x1, y1, z1, x2, y2, z2)
            cnx, cny, cnz, cch = plane(u0, v0, w0, u1, v1, w1, u2, v2, w2)
            pinned = (t0 == _HANDLE) | (t1 == _HANDLE) | (t2 == _HANDLE)
            wf = jnp.where(pinned, _PINNED_W, 1.0)

            for field, val in enumerate((pnx, pny, pnz, pch, cnx, cny, cnz, cch, wf)):
                plsc.store_scatter(
                    obuf, [rows, jnp.full((_L,), field, jnp.int32)], val)
            return carry2

        lax.fori_loop(0, _FA_CHUNK // _L, group, 0)
        pltpu.sync_copy(obuf, ftab_hbm.at[pl.ds(cbase, _FA_CHUNK)])

    for d in idx_copies(0, 0):
        d.start()
    for d in idx_copies(1, 1):
        d.start()
    for d in idx_copies(0, 0):
        d.wait()
    for d in gath_copies(0):
        d.start()

    def pair(t, carry):
        j0 = t * 2
        for b in (0, 1):
            j = j0 + b
            nb = 1 - b
            for d in gath_copies(b):
                d.wait()
            for d in idx_copies(clamp(j + 1), nb):
                d.wait()
            for d in gath_copies(nb):
                d.start()
            for d in idx_copies(clamp(j + 2), b):
                d.start()
            compute(b, base0 + j * _FA_CHUNK)
        return carry

    lax.fori_loop(0, _FA_ITERS // 2, pair, 0)

    for d in gath_copies(0):
        d.wait()
    for d in idx_copies(0, 1):
        d.wait()


def _corr_stage_body(nf_hbm, ft_hbm, ftab_hbm, ntab_hbm, out_hbm,
                     nfa, fta, nfb, ftb, fba, nba, fbb, nbb, acc_v,
                     semIa, semIb, semGa, semGb):
    w = _wid()
    base0 = w * _C_PER_TILE
    lanes = lax.iota(jnp.int32, _L)
    idx = ((nfa, fta), (nfb, ftb))
    gb = ((fba, nba), (fbb, nbb))
    semI = (semIa, semIb)
    semG = (semGa, semGb)
    acc_v[...] = jnp.zeros((_L,), jnp.float32)

    def idx_copies(c, b):
        cbase = base0 + c * _CB_CHUNK
        sl = pl.ds(cbase, _CB_CHUNK)
        return [pltpu.make_async_copy(nf_hbm.at[sl], idx[b][0], semI[b]),
                pltpu.make_async_copy(ft_hbm.at[sl], idx[b][1], semI[b])]

    def gath_copies(b):
        cs = []
        for k in range(_CB_CHUNK // 128):
            sl = pl.ds(k * 128, 128)
            cs.append(pltpu.make_async_copy(
                ftab_hbm.at[idx[b][1].at[sl]], gb[b][0].at[sl], semG[b]))
            cs.append(pltpu.make_async_copy(
                ntab_hbm.at[idx[b][0].at[sl]], gb[b][1].at[sl], semG[b]))
        return cs

    def clamp(c):
        return jnp.minimum(c, _CB_ITERS - 1)

    def compute(b, cbase):
        fbuf, nbuf = gb[b]

        def group(m, carry2):
            rows = m * _L + lanes

            def ldf(field):
                return plsc.load_gather(
                    fbuf, [rows, jnp.full((_L,), field, jnp.int32)])

            def ldn(field):
                return plsc.load_gather(
                    nbuf, [rows, jnp.full((_L,), field, jnp.int32)])

            pnx, pny, pnz, pch = ldf(0), ldf(1), ldf(2), ldf(3)
            cnx, cny, cnz, cch = ldf(4), ldf(5), ldf(6), ldf(7)
            wf = ldf(8)
            px, py, pz = ldn(0), ldn(1), ldn(2)
            vt = ldn(3)
            cx, cy, cz = ldn(4), ldn(5), ldn(6)

            d_prev = px * pnx + py * pny + pz * pnz - pch
            d_curr = cx * cnx + cy * cny + cz * cnz - cch
            stashed = jnp.abs(d_prev) < _CORR_EPS
            dc = d_curr * jnp.sign(d_prev)
            interp = jnp.maximum(_PEN_EPS - dc, 0.0)
            wn = jnp.where(vt == _HANDLE, _PINNED_W, 1.0)
            t = interp * jnp.maximum(wf, wn)
            valid = (cbase + rows) < _C
            acc_v[...] = acc_v[...] + jnp.where(stashed & valid, t * t * t, 0.0)
            return carry2

        lax.fori_loop(0, _CB_CHUNK // _L, group, 0)

    for d in idx_copies(0, 0):
        d.start()
    for d in idx_copies(1, 1):
        d.start()
    for d in idx_copies(0, 0):
        d.wait()
    for d in gath_copies(0):
        d.start()

    def pair(t, carry):
        j0 = t * 2
        for b in (0, 1):
            j = j0 + b
            nb = 1 - b
            for d in gath_copies(b):
                d.wait()
            for d in idx_copies(clamp(j + 1), nb):
                d.wait()
            for d in gath_copies(nb):
                d.start()
            for d in idx_copies(clamp(j + 2), b):
                d.start()
            compute(b, base0 + j * _CB_CHUNK)
        return carry

    lax.fori_loop(0, _CB_ITERS // 2, pair, 0)

    for d in gath_copies(0):
        d.wait()
    for d in idx_copies(0, 1):
        d.wait()

    pltpu.sync_copy(acc_v, out_hbm.at[w])


def _mesh():
    return plsc.VectorSubcoreMesh(core_axis_name="c", subcore_axis_name="s",
                                  num_cores=_NC, num_subcores=_NS)


_face_stage = functools.partial(
    pl.kernel,
    out_type=jax.ShapeDtypeStruct((_F_PAD, 16), jnp.float32),
    mesh=_mesh(),
    compiler_params=pltpu.CompilerParams(
        needs_layout_passes=False, use_tc_tiling_on_sc=False),
    scratch_types=[
        pltpu.VMEM((_FA_CHUNK,), jnp.int32),
        pltpu.VMEM((_FA_CHUNK,), jnp.int32),
        pltpu.VMEM((_FA_CHUNK,), jnp.int32),
        pltpu.VMEM((_FA_CHUNK,), jnp.int32),
        pltpu.VMEM((_FA_CHUNK,), jnp.int32),
        pltpu.VMEM((_FA_CHUNK,), jnp.int32),
        pltpu.VMEM((_FA_CHUNK, 8), jnp.float32),
        pltpu.VMEM((_FA_CHUNK, 8), jnp.float32),
        pltpu.VMEM((_FA_CHUNK, 8), jnp.float32),
        pltpu.VMEM((_FA_CHUNK, 8), jnp.float32),
        pltpu.VMEM((_FA_CHUNK, 8), jnp.float32),
        pltpu.VMEM((_FA_CHUNK, 8), jnp.float32),
        pltpu.VMEM((_FA_CHUNK, 16), jnp.float32),
        pltpu.SemaphoreType.DMA,
        pltpu.SemaphoreType.DMA,
        pltpu.SemaphoreType.DMA,
        pltpu.SemaphoreType.DMA,
    ],
)(_face_stage_body)

_corr_stage = functools.partial(
    pl.kernel,
    out_type=jax.ShapeDtypeStruct((_NW, _L), jnp.float32),
    mesh=_mesh(),
    compiler_params=pltpu.CompilerParams(
        needs_layout_passes=False, use_tc_tiling_on_sc=False),
    scratch_types=[
        pltpu.VMEM((_CB_CHUNK,), jnp.int32),
        pltpu.VMEM((_CB_CHUNK,), jnp.int32),
        pltpu.VMEM((_CB_CHUNK,), jnp.int32),
        pltpu.VMEM((_CB_CHUNK,), jnp.int32),
        pltpu.VMEM((_CB_CHUNK, 16), jnp.float32),
        pltpu.VMEM((_CB_CHUNK, 8), jnp.float32),
        pltpu.VMEM((_CB_CHUNK, 16), jnp.float32),
        pltpu.VMEM((_CB_CHUNK, 8), jnp.float32),
        pltpu.VMEM((_L,), jnp.float32),
        pltpu.SemaphoreType.DMA,
        pltpu.SemaphoreType.DMA,
        pltpu.SemaphoreType.DMA,
        pltpu.SemaphoreType.DMA,
    ],
)(_corr_stage_body)


def kernel(prev_pos, pred_pos, cloth_faces, vertex_type, nodes_from, faces_to, iter_num):
    it = jnp.maximum(iter_num - 0, 0)
    progress = jnp.minimum(it / 1000, 1.0)
    weight = 0.0 + (1.0 - 0.0) * progress

    vt_f = vertex_type.astype(jnp.float32)
    ntab = jnp.concatenate(
        [prev_pos, vt_f, pred_pos, jnp.zeros((_N, 1), jnp.float32)], axis=1)

    f0 = jnp.pad(cloth_faces[:, 0], (0, _F_PAD - _F))
    f1 = jnp.pad(cloth_faces[:, 1], (0, _F_PAD - _F))
    f2 = jnp.pad(cloth_faces[:, 2], (0, _F_PAD - _F))
    nf = jnp.pad(nodes_from[:, 0], (0, _C_PAD - _C))
    ft = jnp.pad(faces_to[:, 0], (0, _C_PAD - _C))

    ftab = _face_stage(ntab, f0, f1, f2)
    partials = _corr_stage(nf, ft, ftab, ntab)

    loss = jnp.sum(partials) * weight
    return (loss, jnp.float32(weight))
